# baseline (speedup 1.0000x reference)
import functools

import jax
import jax.numpy as jnp
from jax import lax
from jax.experimental import pallas as pl
from jax.experimental.pallas import tpu as pltpu
from jax.experimental.pallas import tpu_sc as plsc

NE = 16
K = 2
HD = 1024
FF = 2048
L = 4096
P = L * K
RB = 256
NB = P // RB + NE
NP = NB * RB
FB = 2048
NF = FF // FB
TB = 512

NW = 32
TPW = L // NW
CH = 32


def _sc_wid():
    return lax.axis_index("s") * 2 + lax.axis_index("c")


@functools.cache
def _sc_kernels():
    mesh = plsc.VectorSubcoreMesh(core_axis_name="c", subcore_axis_name="s",
                                  num_cores=2, num_subcores=16)

    @functools.partial(
        pl.kernel,
        out_type=jax.ShapeDtypeStruct((NP, HD), jnp.float32),
        mesh=mesh,
        scratch_types=[
            pltpu.VMEM((CH, HD), jnp.float32),
            pltpu.VMEM((CH,), jnp.int32),
            pltpu.VMEM((CH,), jnp.int32),
            pltpu.SemaphoreType.DMA,
        ],
    )
    def sc_scatter(x_hbm, dest2_hbm, xs_hbm, xbuf, idx0, idx1, sem):
        base = _sc_wid() * TPW

        def chunk(c, carry):
            t0 = base + c * CH
            pltpu.sync_copy(x_hbm.at[pl.ds(t0, CH)], xbuf)
            pltpu.sync_copy(dest2_hbm.at[0, pl.ds(t0, CH)], idx0)
            pltpu.sync_copy(dest2_hbm.at[1, pl.ds(t0, CH)], idx1)
            cp0 = pltpu.async_copy(xbuf, xs_hbm.at[idx0], sem)
            cp1 = pltpu.async_copy(xbuf, xs_hbm.at[idx1], sem)
            cp0.wait()
            cp1.wait()
            return carry

        lax.fori_loop(0, TPW // CH, chunk, 0)

    @functools.partial(
        pl.kernel,
        out_type=jax.ShapeDtypeStruct((L, HD), jnp.float32),
        mesh=mesh,
        scratch_types=[
            pltpu.VMEM((CH, HD), jnp.float32),
            pltpu.VMEM((CH, HD), jnp.float32),
            pltpu.VMEM((CH,), jnp.int32),
            pltpu.VMEM((CH,), jnp.int32),
            pltpu.VMEM((CH,), jnp.float32),
            pltpu.VMEM((CH,), jnp.float32),
            pltpu.SemaphoreType.DMA,
        ],
    )
    def sc_combine(ys_hbm, dest2_hbm, w2_hbm, out_hbm,
                   obuf, y1buf, idx0, idx1, w0v, w1v, sem):
        base = _sc_wid() * TPW

        def chunk(c, carry):
            t0 = base + c * CH
            pltpu.sync_copy(dest2_hbm.at[0, pl.ds(t0, CH)], idx0)
            pltpu.sync_copy(dest2_hbm.at[1, pl.ds(t0, CH)], idx1)
            pltpu.sync_copy(w2_hbm.at[0, pl.ds(t0, CH)], w0v)
            pltpu.sync_copy(w2_hbm.at[1, pl.ds(t0, CH)], w1v)
            cp0 = pltpu.async_copy(ys_hbm.at[idx0], obuf, sem)
            cp1 = pltpu.async_copy(ys_hbm.at[idx1], y1buf, sem)
            cp0.wait()
            cp1.wait()

            for g in range(CH // 16):
                w0g = w0v[pl.ds(g * 16, 16)]
                w1g = w1v[pl.ds(g * 16, 16)]
                for ri in range(16):
                    r = g * 16 + ri
                    w0 = jnp.full((16,), w0g[ri], jnp.float32)
                    w1 = jnp.full((16,), w1g[ri], jnp.float32)

                    @plsc.parallel_loop(0, HD // 16, unroll=4)
                    def col(j, r=r, w0=w0, w1=w1):
                        sl = pl.ds(j * 16, 16)
                        obuf[r, sl] = obuf[r, sl] * w0 + y1buf[r, sl] * w1

            pltpu.sync_copy(obuf, out_hbm.at[pl.ds(t0, CH)])
            return carry

        lax.fori_loop(0, TPW // CH, chunk, 0)

    return sc_scatter, sc_combine


def _sc_scatter(x, dest2):
    return _sc_kernels()[0](x, dest2)


def _sc_combine(ys, dest2, w2):
    return _sc_kernels()[1](ys, dest2, w2)


def _router_body(x_ref, gwt_ref, gb_ref, e1_ref, e2_ref, w1_ref, w2_ref):
    lg = jnp.dot(x_ref[...], gwt_ref[...],
                 preferred_element_type=jnp.float32) + gb_ref[...]
    lanes = jax.lax.broadcasted_iota(jnp.int32, (TB, NE), 1)
    m1 = jnp.max(lg, axis=1, keepdims=True)
    e1 = jnp.min(jnp.where(lg >= m1, lanes, NE), axis=1, keepdims=True)
    lg2 = jnp.where(lanes == e1, -jnp.inf, lg)
    m2 = jnp.max(lg2, axis=1, keepdims=True)
    e2 = jnp.min(jnp.where(lg2 >= m2, lanes, NE), axis=1, keepdims=True)
    s = jnp.sum(jnp.exp(lg - m1), axis=1, keepdims=True)
    e1_ref[...] = e1
    e2_ref[...] = e2
    w1_ref[...] = 1.0 / s
    w2_ref[...] = jnp.exp(m2 - m1) / s


def _router(x, gate_W, gate_b):
    out_shapes = (
        jax.ShapeDtypeStruct((L, 1), jnp.int32),
        jax.ShapeDtypeStruct((L, 1), jnp.int32),
        jax.ShapeDtypeStruct((L, 1), jnp.float32),
        jax.ShapeDtypeStruct((L, 1), jnp.float32),
    )
    spec1 = pl.BlockSpec((TB, 1), lambda b: (b, 0))
    return pl.pallas_call(
        _router_body,
        grid=(L // TB,),
        in_specs=[
            pl.BlockSpec((TB, HD), lambda b: (b, 0)),
            pl.BlockSpec((HD, NE), lambda b: (0, 0)),
            pl.BlockSpec((1, NE), lambda b: (0, 0)),
        ],
        out_specs=(spec1, spec1, spec1, spec1),
        out_shape=out_shapes,
    )(x, gate_W.T, gate_b.reshape(1, NE))


_ER = P // 128


def _dispatch_body(ef_ref, dest_ref, be_ref, nv_ref):
    ef = ef_ref[...]
    lane = jax.lax.broadcasted_iota(jnp.int32, (128, 128), 0)
    lane_t = jax.lax.broadcasted_iota(jnp.int32, (128, 128), 1)
    u_strict = (lane < lane_t).astype(jnp.float32)
    row = jax.lax.broadcasted_iota(jnp.int32, (_ER, _ER), 0)
    row_t = jax.lax.broadcasted_iota(jnp.int32, (_ER, _ER), 1)
    l_strict = (row_t < row).astype(jnp.float32)

    rank = jnp.zeros((_ER, 128), jnp.float32)
    counts = jnp.zeros((1, NE), jnp.float32)
    elane = jax.lax.broadcasted_iota(jnp.int32, (1, NE), 1)
    for e in range(NE):
        m = (ef == e).astype(jnp.float32)
        r1 = jnp.dot(m, u_strict, preferred_element_type=jnp.float32)
        rs = jnp.sum(m, axis=1, keepdims=True)
        pref = jnp.dot(l_strict, rs, preferred_element_type=jnp.float32)
        rank = rank + m * (r1 + pref)
        counts = jnp.where(elane == e, jnp.sum(m), counts)

    bpe = jnp.floor((counts + (RB - 1)) * (1.0 / RB))
    el16 = jax.lax.broadcasted_iota(jnp.int32, (NE, NE), 0)
    el16t = jax.lax.broadcasted_iota(jnp.int32, (NE, NE), 1)
    u16 = (el16 < el16t).astype(jnp.float32)
    bstart = jnp.dot(bpe, u16, preferred_element_type=jnp.float32)
    used = jnp.sum(bpe)
    e_last = jnp.max(jnp.where(counts > 0, elane, -1))

    base = jnp.zeros((_ER, 128), jnp.float32)
    for e in range(NE):
        base = base + (ef == e).astype(jnp.float32) * bstart[0, e]
    dest_ref[...] = (base * RB + rank).astype(jnp.int32)

    bar = jax.lax.broadcasted_iota(jnp.int32, (1, NB), 1).astype(jnp.float32)
    be_acc = jnp.zeros((1, NB), jnp.float32)
    for e in range(NE):
        be_acc = be_acc + (bar >= bstart[0, e]).astype(jnp.float32)
    be_f = jnp.clip(be_acc - 1.0, 0.0, NE - 1.0)
    be_f = jnp.where(bar < used, be_f, e_last.astype(jnp.float32))
    cnt_b = jnp.zeros((1, NB), jnp.float32)
    bst_b = jnp.zeros((1, NB), jnp.float32)
    for e in range(NE):
        sel = (be_f == e).astype(jnp.float32)
        cnt_b = cnt_b + sel * counts[0, e]
        bst_b = bst_b + sel * bstart[0, e]
    nv_f = jnp.clip(cnt_b - (bar - bst_b) * RB, 0.0, RB)
    nv_f = jnp.where(bar < used, nv_f, 0.0)
    be_ref[...] = be_f.astype(jnp.int32)
    nv_ref[...] = nv_f.astype(jnp.int32)


def _dispatch(ef2d):
    return pl.pallas_call(
        _dispatch_body,
        out_shape=(
            jax.ShapeDtypeStruct((_ER, 128), jnp.int32),
            jax.ShapeDtypeStruct((1, NB), jnp.int32),
            jax.ShapeDtypeStruct((1, NB), jnp.int32),
        ),
    )(ef2d)


def _ffn_body(be_ref, nv_ref, xs_ref, wu_ref, bu_ref, wg_ref, bg_ref,
              wd_ref, bd_ref, ys_ref):
    b = pl.program_id(0)
    f = pl.program_id(1)

    @pl.when(nv_ref[b] > 0)
    def _():
        xb = xs_ref[...]
        u = jnp.dot(xb, wu_ref[0], preferred_element_type=jnp.float32) + bu_ref[0]
        g = jnp.dot(xb, wg_ref[0], preferred_element_type=jnp.float32) + bg_ref[0]
        a = u * (g * jax.nn.sigmoid(g))
        y = jnp.dot(a, wd_ref[0], preferred_element_type=jnp.float32)

        @pl.when(f == 0)
        def _():
            ys_ref[...] = y + bd_ref[0, 0]

        @pl.when(f != 0)
        def _():
            ys_ref[...] = ys_ref[...] + y


def _expert_ffn(be, nv, xs, Wu, bu, Wg, bg, Wd, bd):
    grid_spec = pltpu.PrefetchScalarGridSpec(
        num_scalar_prefetch=2,
        grid=(NB, NF),
        in_specs=[
            pl.BlockSpec((RB, HD), lambda b, f, be, nv: (b * (nv[b] > 0), 0)),
            pl.BlockSpec((1, HD, FB),
                         lambda b, f, be, nv: (be[b], 0, f * (nv[b] > 0))),
            pl.BlockSpec((1, 1, FB),
                         lambda b, f, be, nv: (be[b], 0, f * (nv[b] > 0))),
            pl.BlockSpec((1, HD, FB),
                         lambda b, f, be, nv: (be[b], 0, f * (nv[b] > 0))),
            pl.BlockSpec((1, 1, FB),
                         lambda b, f, be, nv: (be[b], 0, f * (nv[b] > 0))),
            pl.BlockSpec((1, FB, HD),
                         lambda b, f, be, nv: (be[b], f * (nv[b] > 0), 0)),
            pl.BlockSpec((1, 1, HD), lambda b, f, be, nv: (be[b], 0, 0)),
        ],
        out_specs=pl.BlockSpec((RB, HD), lambda b, f, be, nv: (b, 0)),
    )
    return pl.pallas_call(
        _ffn_body,
        grid_spec=grid_spec,
        out_shape=jax.ShapeDtypeStruct((NP, HD), jnp.float32),
        compiler_params=pltpu.CompilerParams(
            dimension_semantics=("arbitrary", "arbitrary"),
        ),
    )(be, nv, xs, Wu, bu.reshape(NE, 1, FF), Wg, bg.reshape(NE, 1, FF),
      Wd, bd.reshape(NE, 1, HD))


def kernel(x, gate_W, gate_b, Wu, bu, Wg, bg, Wd, bd):
    e1, e2, w1, w2v = _router(x, gate_W, gate_b)
    e_flat = jnp.concatenate([e1, e2], axis=1).reshape(-1)
    w2 = jnp.concatenate([w1, w2v], axis=1).T

    dest2d, be2d, nv2d = _dispatch(e_flat.reshape(_ER, 128))
    be = be2d.reshape(NB)
    nv = nv2d.reshape(NB)

    dest2 = dest2d.reshape(L, K).T
    xs = _sc_scatter(x, dest2)

    ys = _expert_ffn(be, nv, xs, Wu, bu, Wg, bg, Wd, bd)

    return _sc_combine(ys, dest2, w2)


# speedup vs baseline: 1.0373x; 1.0373x over previous
import functools

import jax
import jax.numpy as jnp
from jax import lax
from jax.experimental import pallas as pl
from jax.experimental.pallas import tpu as pltpu
from jax.experimental.pallas import tpu_sc as plsc

NE = 16
K = 2
HD = 1024
FF = 2048
L = 4096
P = L * K
RB = 512
NB = P // RB + NE
NP = NB * RB
FB = 1024
NF = FF // FB
TB = 512

NW = 32
TPW = L // NW
CH = 32


def _sc_wid():
    return lax.axis_index("s") * 2 + lax.axis_index("c")


@functools.cache
def _sc_kernels():
    mesh = plsc.VectorSubcoreMesh(core_axis_name="c", subcore_axis_name="s",
                                  num_cores=2, num_subcores=16)

    @functools.partial(
        pl.kernel,
        out_type=jax.ShapeDtypeStruct((NP, HD), jnp.float32),
        mesh=mesh,
        scratch_types=[
            pltpu.VMEM((CH, HD), jnp.float32),
            pltpu.VMEM((CH,), jnp.int32),
            pltpu.VMEM((CH,), jnp.int32),
            pltpu.SemaphoreType.DMA,
        ],
    )
    def sc_scatter(x_hbm, dest2_hbm, xs_hbm, xbuf, idx0, idx1, sem):
        base = _sc_wid() * TPW

        def chunk(c, carry):
            t0 = base + c * CH
            pltpu.sync_copy(x_hbm.at[pl.ds(t0, CH)], xbuf)
            pltpu.sync_copy(dest2_hbm.at[0, pl.ds(t0, CH)], idx0)
            pltpu.sync_copy(dest2_hbm.at[1, pl.ds(t0, CH)], idx1)
            cp0 = pltpu.async_copy(xbuf, xs_hbm.at[idx0], sem)
            cp1 = pltpu.async_copy(xbuf, xs_hbm.at[idx1], sem)
            cp0.wait()
            cp1.wait()
            return carry

        lax.fori_loop(0, TPW // CH, chunk, 0)

    @functools.partial(
        pl.kernel,
        out_type=jax.ShapeDtypeStruct((L, HD), jnp.float32),
        mesh=mesh,
        scratch_types=[
            pltpu.VMEM((CH, HD), jnp.float32),
            pltpu.VMEM((CH, HD), jnp.float32),
            pltpu.VMEM((CH,), jnp.int32),
            pltpu.VMEM((CH,), jnp.int32),
            pltpu.VMEM((CH,), jnp.float32),
            pltpu.VMEM((CH,), jnp.float32),
            pltpu.SemaphoreType.DMA,
        ],
    )
    def sc_combine(ys_hbm, dest2_hbm, w2_hbm, out_hbm,
                   obuf, y1buf, idx0, idx1, w0v, w1v, sem):
        base = _sc_wid() * TPW

        def chunk(c, carry):
            t0 = base + c * CH
            pltpu.sync_copy(dest2_hbm.at[0, pl.ds(t0, CH)], idx0)
            pltpu.sync_copy(dest2_hbm.at[1, pl.ds(t0, CH)], idx1)
            pltpu.sync_copy(w2_hbm.at[0, pl.ds(t0, CH)], w0v)
            pltpu.sync_copy(w2_hbm.at[1, pl.ds(t0, CH)], w1v)
            cp0 = pltpu.async_copy(ys_hbm.at[idx0], obuf, sem)
            cp1 = pltpu.async_copy(ys_hbm.at[idx1], y1buf, sem)
            cp0.wait()
            cp1.wait()

            for g in range(CH // 16):
                w0g = w0v[pl.ds(g * 16, 16)]
                w1g = w1v[pl.ds(g * 16, 16)]
                for ri in range(16):
                    r = g * 16 + ri
                    w0 = jnp.full((16,), w0g[ri], jnp.float32)
                    w1 = jnp.full((16,), w1g[ri], jnp.float32)

                    @plsc.parallel_loop(0, HD // 16, unroll=4)
                    def col(j, r=r, w0=w0, w1=w1):
                        sl = pl.ds(j * 16, 16)
                        obuf[r, sl] = obuf[r, sl] * w0 + y1buf[r, sl] * w1

            pltpu.sync_copy(obuf, out_hbm.at[pl.ds(t0, CH)])
            return carry

        lax.fori_loop(0, TPW // CH, chunk, 0)

    return sc_scatter, sc_combine


def _sc_scatter(x, dest2):
    return _sc_kernels()[0](x, dest2)


def _sc_combine(ys, dest2, w2):
    return _sc_kernels()[1](ys, dest2, w2)


def _router_body(x_ref, gwt_ref, gb_ref, e1_ref, e2_ref, w1_ref, w2_ref):
    lg = jnp.dot(x_ref[...], gwt_ref[...],
                 preferred_element_type=jnp.float32) + gb_ref[...]
    lanes = jax.lax.broadcasted_iota(jnp.int32, (TB, NE), 1)
    m1 = jnp.max(lg, axis=1, keepdims=True)
    e1 = jnp.min(jnp.where(lg >= m1, lanes, NE), axis=1, keepdims=True)
    lg2 = jnp.where(lanes == e1, -jnp.inf, lg)
    m2 = jnp.max(lg2, axis=1, keepdims=True)
    e2 = jnp.min(jnp.where(lg2 >= m2, lanes, NE), axis=1, keepdims=True)
    s = jnp.sum(jnp.exp(lg - m1), axis=1, keepdims=True)
    e1_ref[...] = e1
    e2_ref[...] = e2
    w1_ref[...] = 1.0 / s
    w2_ref[...] = jnp.exp(m2 - m1) / s


def _router(x, gate_W, gate_b):
    out_shapes = (
        jax.ShapeDtypeStruct((L, 1), jnp.int32),
        jax.ShapeDtypeStruct((L, 1), jnp.int32),
        jax.ShapeDtypeStruct((L, 1), jnp.float32),
        jax.ShapeDtypeStruct((L, 1), jnp.float32),
    )
    spec1 = pl.BlockSpec((TB, 1), lambda b: (b, 0))
    return pl.pallas_call(
        _router_body,
        grid=(L // TB,),
        in_specs=[
            pl.BlockSpec((TB, HD), lambda b: (b, 0)),
            pl.BlockSpec((HD, NE), lambda b: (0, 0)),
            pl.BlockSpec((1, NE), lambda b: (0, 0)),
        ],
        out_specs=(spec1, spec1, spec1, spec1),
        out_shape=out_shapes,
    )(x, gate_W.T, gate_b.reshape(1, NE))


_ER = P // 128


def _dispatch_body(ef_ref, dest_ref, be_ref, nv_ref):
    ef = ef_ref[...]
    lane = jax.lax.broadcasted_iota(jnp.int32, (128, 128), 0)
    lane_t = jax.lax.broadcasted_iota(jnp.int32, (128, 128), 1)
    u_strict = (lane < lane_t).astype(jnp.float32)
    row = jax.lax.broadcasted_iota(jnp.int32, (_ER, _ER), 0)
    row_t = jax.lax.broadcasted_iota(jnp.int32, (_ER, _ER), 1)
    l_strict = (row_t < row).astype(jnp.float32)

    rank = jnp.zeros((_ER, 128), jnp.float32)
    counts = jnp.zeros((1, NE), jnp.float32)
    elane = jax.lax.broadcasted_iota(jnp.int32, (1, NE), 1)
    for e in range(NE):
        m = (ef == e).astype(jnp.float32)
        r1 = jnp.dot(m, u_strict, preferred_element_type=jnp.float32)
        rs = jnp.sum(m, axis=1, keepdims=True)
        pref = jnp.dot(l_strict, rs, preferred_element_type=jnp.float32)
        rank = rank + m * (r1 + pref)
        counts = jnp.where(elane == e, jnp.sum(m), counts)

    bpe = jnp.floor((counts + (RB - 1)) * (1.0 / RB))
    el16 = jax.lax.broadcasted_iota(jnp.int32, (NE, NE), 0)
    el16t = jax.lax.broadcasted_iota(jnp.int32, (NE, NE), 1)
    u16 = (el16 < el16t).astype(jnp.float32)
    bstart = jnp.dot(bpe, u16, preferred_element_type=jnp.float32)
    used = jnp.sum(bpe)
    e_last = jnp.max(jnp.where(counts > 0, elane, -1))

    base = jnp.zeros((_ER, 128), jnp.float32)
    for e in range(NE):
        base = base + (ef == e).astype(jnp.float32) * bstart[0, e]
    dest_ref[...] = (base * RB + rank).astype(jnp.int32)

    bar = jax.lax.broadcasted_iota(jnp.int32, (1, NB), 1).astype(jnp.float32)
    be_acc = jnp.zeros((1, NB), jnp.float32)
    for e in range(NE):
        be_acc = be_acc + (bar >= bstart[0, e]).astype(jnp.float32)
    be_f = jnp.clip(be_acc - 1.0, 0.0, NE - 1.0)
    be_f = jnp.where(bar < used, be_f, e_last.astype(jnp.float32))
    cnt_b = jnp.zeros((1, NB), jnp.float32)
    bst_b = jnp.zeros((1, NB), jnp.float32)
    for e in range(NE):
        sel = (be_f == e).astype(jnp.float32)
        cnt_b = cnt_b + sel * counts[0, e]
        bst_b = bst_b + sel * bstart[0, e]
    nv_f = jnp.clip(cnt_b - (bar - bst_b) * RB, 0.0, RB)
    nv_f = jnp.where(bar < used, nv_f, 0.0)
    be_ref[...] = be_f.astype(jnp.int32)
    nv_ref[...] = nv_f.astype(jnp.int32)


def _dispatch(ef2d):
    return pl.pallas_call(
        _dispatch_body,
        out_shape=(
            jax.ShapeDtypeStruct((_ER, 128), jnp.int32),
            jax.ShapeDtypeStruct((1, NB), jnp.int32),
            jax.ShapeDtypeStruct((1, NB), jnp.int32),
        ),
    )(ef2d)


def _ffn_body(be_ref, nv_ref, xs_ref, wu_ref, bu_ref, wg_ref, bg_ref,
              wd_ref, bd_ref, ys_ref):
    b = pl.program_id(0)
    f = pl.program_id(1)

    @pl.when(nv_ref[b] > 0)
    def _():
        xb = xs_ref[...]
        u = jnp.dot(xb, wu_ref[0], preferred_element_type=jnp.float32) + bu_ref[0]
        g = jnp.dot(xb, wg_ref[0], preferred_element_type=jnp.float32) + bg_ref[0]
        a = u * (g * jax.nn.sigmoid(g))
        y = jnp.dot(a, wd_ref[0], preferred_element_type=jnp.float32)

        @pl.when(f == 0)
        def _():
            ys_ref[...] = y + bd_ref[0, 0]

        @pl.when(f != 0)
        def _():
            ys_ref[...] = ys_ref[...] + y


def _expert_ffn(be, nv, xs, Wu, bu, Wg, bg, Wd, bd):
    grid_spec = pltpu.PrefetchScalarGridSpec(
        num_scalar_prefetch=2,
        grid=(NB, NF),
        in_specs=[
            pl.BlockSpec((RB, HD), lambda b, f, be, nv: (b * (nv[b] > 0), 0)),
            pl.BlockSpec((1, HD, FB),
                         lambda b, f, be, nv: (be[b], 0, f * (nv[b] > 0))),
            pl.BlockSpec((1, 1, FB),
                         lambda b, f, be, nv: (be[b], 0, f * (nv[b] > 0))),
            pl.BlockSpec((1, HD, FB),
                         lambda b, f, be, nv: (be[b], 0, f * (nv[b] > 0))),
            pl.BlockSpec((1, 1, FB),
                         lambda b, f, be, nv: (be[b], 0, f * (nv[b] > 0))),
            pl.BlockSpec((1, FB, HD),
                         lambda b, f, be, nv: (be[b], f * (nv[b] > 0), 0)),
            pl.BlockSpec((1, 1, HD), lambda b, f, be, nv: (be[b], 0, 0)),
        ],
        out_specs=pl.BlockSpec((RB, HD), lambda b, f, be, nv: (b, 0)),
    )
    return pl.pallas_call(
        _ffn_body,
        grid_spec=grid_spec,
        out_shape=jax.ShapeDtypeStruct((NP, HD), jnp.float32),
        compiler_params=pltpu.CompilerParams(
            dimension_semantics=("arbitrary", "arbitrary"),
        ),
    )(be, nv, xs, Wu, bu.reshape(NE, 1, FF), Wg, bg.reshape(NE, 1, FF),
      Wd, bd.reshape(NE, 1, HD))


def kernel(x, gate_W, gate_b, Wu, bu, Wg, bg, Wd, bd):
    e1, e2, w1, w2v = _router(x, gate_W, gate_b)
    e_flat = jnp.concatenate([e1, e2], axis=0).reshape(-1)
    w2 = jnp.concatenate([w1.reshape(1, L), w2v.reshape(1, L)], axis=0)

    dest2d, be2d, nv2d = _dispatch(e_flat.reshape(_ER, 128))
    be = be2d.reshape(NB)
    nv = nv2d.reshape(NB)

    dest2 = dest2d.reshape(K, L)
    xs = _sc_scatter(x, dest2)

    ys = _expert_ffn(be, nv, xs, Wu, bu, Wg, bg, Wd, bd)

    return _sc_combine(ys, dest2, w2)


# speedup vs baseline: 1.0381x; 1.0008x over previous
import functools

import jax
import jax.numpy as jnp
from jax import lax
from jax.experimental import pallas as pl
from jax.experimental.pallas import tpu as pltpu
from jax.experimental.pallas import tpu_sc as plsc

NE = 16
K = 2
HD = 1024
FF = 2048
L = 4096
P = L * K
RB = 512
NB = P // RB + NE
NP = NB * RB
FB = 1024
NF = FF // FB
TB = 512

NW = 32
TPW = L // NW
CH = 32


def _sc_wid():
    return lax.axis_index("s") * 2 + lax.axis_index("c")


@functools.cache
def _sc_kernels():
    mesh = plsc.VectorSubcoreMesh(core_axis_name="c", subcore_axis_name="s",
                                  num_cores=2, num_subcores=16)

    @functools.partial(
        pl.kernel,
        out_type=jax.ShapeDtypeStruct((NP, HD), jnp.float32),
        mesh=mesh,
        scratch_types=[
            pltpu.VMEM((CH, HD), jnp.float32),
            pltpu.VMEM((CH,), jnp.int32),
            pltpu.VMEM((CH,), jnp.int32),
            pltpu.SemaphoreType.DMA,
        ],
    )
    def sc_scatter(x_hbm, dest2_hbm, xs_hbm, xbuf, idx0, idx1, sem):
        base = _sc_wid() * TPW

        def chunk(c, carry):
            t0 = base + c * CH
            pltpu.sync_copy(x_hbm.at[pl.ds(t0, CH)], xbuf)
            pltpu.sync_copy(dest2_hbm.at[0, pl.ds(t0, CH)], idx0)
            pltpu.sync_copy(dest2_hbm.at[1, pl.ds(t0, CH)], idx1)
            cp0 = pltpu.async_copy(xbuf, xs_hbm.at[idx0], sem)
            cp1 = pltpu.async_copy(xbuf, xs_hbm.at[idx1], sem)
            cp0.wait()
            cp1.wait()
            return carry

        lax.fori_loop(0, TPW // CH, chunk, 0)

    @functools.partial(
        pl.kernel,
        out_type=jax.ShapeDtypeStruct((L, HD), jnp.float32),
        mesh=mesh,
        scratch_types=[
            pltpu.VMEM((CH, HD), jnp.float32),
            pltpu.VMEM((CH, HD), jnp.float32),
            pltpu.VMEM((CH,), jnp.int32),
            pltpu.VMEM((CH,), jnp.int32),
            pltpu.VMEM((CH,), jnp.float32),
            pltpu.VMEM((CH,), jnp.float32),
            pltpu.SemaphoreType.DMA,
        ],
    )
    def sc_combine(ys_hbm, dest2_hbm, w2_hbm, out_hbm,
                   obuf, y1buf, idx0, idx1, w0v, w1v, sem):
        base = _sc_wid() * TPW

        def chunk(c, carry):
            t0 = base + c * CH
            pltpu.sync_copy(dest2_hbm.at[0, pl.ds(t0, CH)], idx0)
            pltpu.sync_copy(dest2_hbm.at[1, pl.ds(t0, CH)], idx1)
            pltpu.sync_copy(w2_hbm.at[0, pl.ds(t0, CH)], w0v)
            pltpu.sync_copy(w2_hbm.at[1, pl.ds(t0, CH)], w1v)
            cp0 = pltpu.async_copy(ys_hbm.at[idx0], obuf, sem)
            cp1 = pltpu.async_copy(ys_hbm.at[idx1], y1buf, sem)
            cp0.wait()
            cp1.wait()

            for g in range(CH // 16):
                w0g = w0v[pl.ds(g * 16, 16)]
                w1g = w1v[pl.ds(g * 16, 16)]
                for ri in range(16):
                    r = g * 16 + ri
                    w0 = jnp.full((16,), w0g[ri], jnp.float32)
                    w1 = jnp.full((16,), w1g[ri], jnp.float32)

                    @plsc.parallel_loop(0, HD // 16, unroll=4)
                    def col(j, r=r, w0=w0, w1=w1):
                        sl = pl.ds(j * 16, 16)
                        obuf[r, sl] = obuf[r, sl] * w0 + y1buf[r, sl] * w1

            pltpu.sync_copy(obuf, out_hbm.at[pl.ds(t0, CH)])
            return carry

        lax.fori_loop(0, TPW // CH, chunk, 0)

    return sc_scatter, sc_combine


def _sc_scatter(x, dest2):
    return _sc_kernels()[0](x, dest2)


def _sc_combine(ys, dest2, w2):
    return _sc_kernels()[1](ys, dest2, w2)


def _router_body(x_ref, gwt_ref, gb_ref, e1_ref, e2_ref, w1_ref, w2_ref):
    lg = jnp.dot(x_ref[...], gwt_ref[...],
                 preferred_element_type=jnp.float32) + gb_ref[...]
    lanes = jax.lax.broadcasted_iota(jnp.int32, (TB, NE), 1)
    m = jnp.max(lg, axis=1, keepdims=True)
    ex = jnp.exp(lg - m)
    pr = ex / jnp.sum(ex, axis=1, keepdims=True)
    p1 = jnp.max(pr, axis=1, keepdims=True)
    e1 = jnp.min(jnp.where(pr >= p1, lanes, NE), axis=1, keepdims=True)
    pr2 = jnp.where(lanes == e1, -jnp.inf, pr)
    p2 = jnp.max(pr2, axis=1, keepdims=True)
    e2 = jnp.min(jnp.where(pr2 >= p2, lanes, NE), axis=1, keepdims=True)
    e1_ref[...] = e1
    e2_ref[...] = e2
    w1_ref[...] = p1
    w2_ref[...] = p2


def _router(x, gate_W, gate_b):
    out_shapes = (
        jax.ShapeDtypeStruct((L, 1), jnp.int32),
        jax.ShapeDtypeStruct((L, 1), jnp.int32),
        jax.ShapeDtypeStruct((L, 1), jnp.float32),
        jax.ShapeDtypeStruct((L, 1), jnp.float32),
    )
    spec1 = pl.BlockSpec((TB, 1), lambda b: (b, 0))
    return pl.pallas_call(
        _router_body,
        grid=(L // TB,),
        in_specs=[
            pl.BlockSpec((TB, HD), lambda b: (b, 0)),
            pl.BlockSpec((HD, NE), lambda b: (0, 0)),
            pl.BlockSpec((1, NE), lambda b: (0, 0)),
        ],
        out_specs=(spec1, spec1, spec1, spec1),
        out_shape=out_shapes,
    )(x, gate_W.T, gate_b.reshape(1, NE))


_ER = P // 128


def _dispatch_body(ef_ref, dest_ref, be_ref, nv_ref):
    ef = ef_ref[...]
    lane = jax.lax.broadcasted_iota(jnp.int32, (128, 128), 0)
    lane_t = jax.lax.broadcasted_iota(jnp.int32, (128, 128), 1)
    u_strict = (lane < lane_t).astype(jnp.float32)
    row = jax.lax.broadcasted_iota(jnp.int32, (_ER, _ER), 0)
    row_t = jax.lax.broadcasted_iota(jnp.int32, (_ER, _ER), 1)
    l_strict = (row_t < row).astype(jnp.float32)

    rank = jnp.zeros((_ER, 128), jnp.float32)
    counts = jnp.zeros((1, NE), jnp.float32)
    elane = jax.lax.broadcasted_iota(jnp.int32, (1, NE), 1)
    for e in range(NE):
        m = (ef == e).astype(jnp.float32)
        r1 = jnp.dot(m, u_strict, preferred_element_type=jnp.float32)
        rs = jnp.sum(m, axis=1, keepdims=True)
        pref = jnp.dot(l_strict, rs, preferred_element_type=jnp.float32)
        rank = rank + m * (r1 + pref)
        counts = jnp.where(elane == e, jnp.sum(m), counts)

    bpe = jnp.floor((counts + (RB - 1)) * (1.0 / RB))
    el16 = jax.lax.broadcasted_iota(jnp.int32, (NE, NE), 0)
    el16t = jax.lax.broadcasted_iota(jnp.int32, (NE, NE), 1)
    u16 = (el16 < el16t).astype(jnp.float32)
    bstart = jnp.dot(bpe, u16, preferred_element_type=jnp.float32)
    used = jnp.sum(bpe)
    e_last = jnp.max(jnp.where(counts > 0, elane, -1))

    base = jnp.zeros((_ER, 128), jnp.float32)
    for e in range(NE):
        base = base + (ef == e).astype(jnp.float32) * bstart[0, e]
    dest_ref[...] = (base * RB + rank).astype(jnp.int32)

    bar = jax.lax.broadcasted_iota(jnp.int32, (1, NB), 1).astype(jnp.float32)
    be_acc = jnp.zeros((1, NB), jnp.float32)
    for e in range(NE):
        be_acc = be_acc + (bar >= bstart[0, e]).astype(jnp.float32)
    be_f = jnp.clip(be_acc - 1.0, 0.0, NE - 1.0)
    be_f = jnp.where(bar < used, be_f, e_last.astype(jnp.float32))
    cnt_b = jnp.zeros((1, NB), jnp.float32)
    bst_b = jnp.zeros((1, NB), jnp.float32)
    for e in range(NE):
        sel = (be_f == e).astype(jnp.float32)
        cnt_b = cnt_b + sel * counts[0, e]
        bst_b = bst_b + sel * bstart[0, e]
    nv_f = jnp.clip(cnt_b - (bar - bst_b) * RB, 0.0, RB)
    nv_f = jnp.where(bar < used, nv_f, 0.0)
    be_ref[...] = be_f.astype(jnp.int32)
    nv_ref[...] = nv_f.astype(jnp.int32)


def _dispatch(ef2d):
    return pl.pallas_call(
        _dispatch_body,
        out_shape=(
            jax.ShapeDtypeStruct((_ER, 128), jnp.int32),
            jax.ShapeDtypeStruct((1, NB), jnp.int32),
            jax.ShapeDtypeStruct((1, NB), jnp.int32),
        ),
    )(ef2d)


def _ffn_body(be_ref, nv_ref, xs_ref, wu_ref, bu_ref, wg_ref, bg_ref,
              wd_ref, bd_ref, ys_ref):
    b = pl.program_id(0)
    f = pl.program_id(1)

    @pl.when(nv_ref[b] > 0)
    def _():
        xb = xs_ref[...]
        u = jnp.dot(xb, wu_ref[0], preferred_element_type=jnp.float32) + bu_ref[0]
        g = jnp.dot(xb, wg_ref[0], preferred_element_type=jnp.float32) + bg_ref[0]
        a = u * (g * jax.nn.sigmoid(g))
        y = jnp.dot(a, wd_ref[0], preferred_element_type=jnp.float32)

        @pl.when(f == 0)
        def _():
            ys_ref[...] = y + bd_ref[0, 0]

        @pl.when(f != 0)
        def _():
            ys_ref[...] = ys_ref[...] + y


def _expert_ffn(be, nv, xs, Wu, bu, Wg, bg, Wd, bd):
    grid_spec = pltpu.PrefetchScalarGridSpec(
        num_scalar_prefetch=2,
        grid=(NB, NF),
        in_specs=[
            pl.BlockSpec((RB, HD), lambda b, f, be, nv: (b * (nv[b] > 0), 0)),
            pl.BlockSpec((1, HD, FB),
                         lambda b, f, be, nv: (be[b], 0, f * (nv[b] > 0))),
            pl.BlockSpec((1, 1, FB),
                         lambda b, f, be, nv: (be[b], 0, f * (nv[b] > 0))),
            pl.BlockSpec((1, HD, FB),
                         lambda b, f, be, nv: (be[b], 0, f * (nv[b] > 0))),
            pl.BlockSpec((1, 1, FB),
                         lambda b, f, be, nv: (be[b], 0, f * (nv[b] > 0))),
            pl.BlockSpec((1, FB, HD),
                         lambda b, f, be, nv: (be[b], f * (nv[b] > 0), 0)),
            pl.BlockSpec((1, 1, HD), lambda b, f, be, nv: (be[b], 0, 0)),
        ],
        out_specs=pl.BlockSpec((RB, HD), lambda b, f, be, nv: (b, 0)),
    )
    return pl.pallas_call(
        _ffn_body,
        grid_spec=grid_spec,
        out_shape=jax.ShapeDtypeStruct((NP, HD), jnp.float32),
        compiler_params=pltpu.CompilerParams(
            dimension_semantics=("arbitrary", "arbitrary"),
        ),
    )(be, nv, xs, Wu, bu.reshape(NE, 1, FF), Wg, bg.reshape(NE, 1, FF),
      Wd, bd.reshape(NE, 1, HD))


def kernel(x, gate_W, gate_b, Wu, bu, Wg, bg, Wd, bd):
    e1, e2, w1, w2v = _router(x, gate_W, gate_b)
    e_flat = jnp.concatenate([e1, e2], axis=0).reshape(-1)
    w2 = jnp.concatenate([w1.reshape(1, L), w2v.reshape(1, L)], axis=0)

    dest2d, be2d, nv2d = _dispatch(e_flat.reshape(_ER, 128))
    be = be2d.reshape(NB)
    nv = nv2d.reshape(NB)

    dest2 = dest2d.reshape(K, L)
    xs = _sc_scatter(x, dest2)

    ys = _expert_ffn(be, nv, xs, Wu, bu, Wg, bg, Wd, bd)

    return _sc_combine(ys, dest2, w2)
